# baseline (speedup 1.0000x reference)
import jax
import jax.numpy as jnp
from jax.experimental import pallas as pl
from jax.experimental.pallas import tpu as pltpu

_B, _N, _M = 8, 2048, 2048
_TI = 256
_NI = _N // _TI


def _chamfer_tc_kernel(a_ref, bt_ref, out_ref, colmin_ref):
    b_i = pl.program_id(0)
    i = pl.program_id(1)

    a = a_ref[0]
    bt = bt_ref[0]
    a2 = jnp.sum(a * a, axis=1, keepdims=True)
    b2 = jnp.sum(bt * bt, axis=0, keepdims=True)
    mm = jax.lax.dot_general(
        a.astype(jnp.bfloat16), bt.astype(jnp.bfloat16),
        (((1,), (0,)), ((), ())),
        preferred_element_type=jnp.float32)
    d = a2 + b2 - 2.0 * mm

    rowmin = jnp.min(d, axis=1)
    colmin = jnp.min(d, axis=0, keepdims=True)

    @pl.when(jnp.logical_and(b_i == 0, i == 0))
    def _():
        out_ref[0] = 0.0

    @pl.when(i == 0)
    def _():
        colmin_ref[...] = colmin

    @pl.when(i != 0)
    def _():
        colmin_ref[...] = jnp.minimum(colmin_ref[...], colmin)

    out_ref[0] += jnp.sum(rowmin) * (1.0 / (_B * _N))

    @pl.when(i == _NI - 1)
    def _():
        out_ref[0] += jnp.sum(colmin_ref[...]) * (1.0 / (_B * _M))


def kernel(p1, p2):
    p2t = jnp.transpose(p2, (0, 2, 1))
    out = pl.pallas_call(
        _chamfer_tc_kernel,
        grid=(_B, _NI),
        in_specs=[
            pl.BlockSpec((1, _TI, 3), lambda b, i: (b, i, 0)),
            pl.BlockSpec((1, 3, _M), lambda b, i: (b, 0, 0)),
        ],
        out_specs=pl.BlockSpec(memory_space=pltpu.SMEM),
        out_shape=jax.ShapeDtypeStruct((1,), jnp.float32),
        scratch_shapes=[pltpu.VMEM((1, _M), jnp.float32)],
    )(p1, p2t)
    return out
